# baseline (speedup 1.0000x reference)
import functools

import jax
import jax.numpy as jnp
from jax import lax
from jax.experimental import pallas as pl
from jax.experimental.pallas import tpu as pltpu
from jax.experimental.pallas import tpu_sc as plsc

N = 50000
D = 128
P = 64
HALF = N // 2
BN = 1000
GRIDH = HALF // BN
NC = 2
NS = 16
NW = NC * NS
C = 128
CAP = 896
NCH = CAP // C
NPAD = 29000
NSEG = 2 * P


def _tc_kernel(feat_ref, geo_ref, cl8_ref, prm_ref,
               w1_ref, w2_ref, w3_ref, b1_ref, b2_ref, b3_ref, pw_ref,
               pl_ref, h2_ref, hw_ref):
    c8 = cl8_ref[...]
    prm = prm_ref[...]
    x = c8[:, 0:1]
    y = c8[:, 1:2]
    z = c8[:, 2:3]
    rmask = ((x >= prm[0:1, :]) & (x < prm[1:2, :]) &
             (y >= prm[2:3, :]) & (y < prm[3:4, :]))
    bf = lambda a: a.astype(jnp.bfloat16).astype(jnp.float32)
    dx = bf(x - prm[4:5, :])
    dy = bf(y - prm[5:6, :])
    dz = bf(z - prm[6:7, :])
    dist = jnp.abs(dx * bf(prm[7:8, :]) + dy * bf(prm[8:9, :])
                   + dz * bf(prm[9:10, :]))
    mask = rmask & (dist < 0.1)

    fcat = jnp.concatenate([feat_ref[...], geo_ref[...]], axis=1)
    h1 = jnp.maximum(
        jnp.dot(fcat, w1_ref[...], preferred_element_type=jnp.float32)
        + b1_ref[...], 0.0)
    h2 = jnp.maximum(
        jnp.dot(h1, w2_ref[...], preferred_element_type=jnp.float32)
        + b2_ref[...], 0.0)
    logit = (jnp.dot(h2, w3_ref[...], preferred_element_type=jnp.float32)
             + b3_ref[...])

    pl_ref[...] = jnp.where(mask, logit, 0.0)
    h2_ref[...] = h2

    pos = logit > 0.0
    mo = jnp.concatenate([(mask & pos).astype(jnp.float32),
                          (mask & (~pos)).astype(jnp.float32)], axis=1)
    hw_f = jnp.dot(mo, pw_ref[...], preferred_element_type=jnp.float32)
    hw_ref[...] = hw_f.astype(jnp.int32)


def _merge_kernel(acc_a_ref, acc_b_ref, on_ref, off_ref):
    m = jnp.maximum(acc_a_ref[0], acc_b_ref[0])
    for t in range(1, NW):
        m = jnp.maximum(m, jnp.maximum(acc_a_ref[t], acc_b_ref[t]))
    on_ref[...] = m[:P]
    off_ref[...] = m[P:]


def _make_sc_seg_max():
    mesh = plsc.VectorSubcoreMesh(core_axis_name="c", subcore_axis_name="s")

    @functools.partial(
        pl.kernel, mesh=mesh,
        out_type=jax.ShapeDtypeStruct((NW, NSEG, D), jnp.float32),
        scratch_types=[
            pltpu.VMEM((2, C, D), jnp.float32),
            pltpu.VMEM((2, C, 16), jnp.int32),
            pltpu.VMEM((NSEG, D), jnp.float32),
            pltpu.SemaphoreType.DMA((4,)),
        ],
    )
    def sc_seg_max(h2_hbm, hw_hbm, out_hbm, h2_v, hw_v, acc_v, sems):
        wid = lax.axis_index("s") * NC + lax.axis_index("c")
        base = wid * CAP

        def _zero_row(r, _):
            for j in range(D // 16):
                acc_v[r, pl.ds(16 * j, 16)] = jnp.zeros((16,), jnp.float32)
            return 0

        lax.fori_loop(0, NSEG, _zero_row, 0)

        def _issue(c, b):
            start = base + c * C
            return (pltpu.async_copy(h2_hbm.at[pl.ds(start, C)],
                                     h2_v.at[b], sems.at[2 * b]),
                    pltpu.async_copy(hw_hbm.at[pl.ds(start, C)],
                                     hw_v.at[b], sems.at[2 * b + 1]))

        pending = _issue(0, 0)
        for c in range(NCH):
            b = c % 2
            for h in pending:
                h.wait()
            if c + 1 < NCH:
                pending = _issue(c + 1, 1 - b)
            start = base + c * C

            def _point(ci, _):
                valid = (start + ci) < HALF
                row = [h2_v[b, ci, pl.ds(16 * j, 16)] for j in range(D // 16)]
                hwrow = hw_v[b, ci, pl.ds(0, 16)]

                for g in range(4):
                    w32 = hwrow[2 * g] | (hwrow[2 * g + 1] << 16)
                    w0 = jnp.where(valid, w32, 0)
                    cnt = jnp.where(valid, hwrow[8 + g], 0)

                    @pl.loop(0, cnt, init_carry=w0, unroll=False)
                    def _walk(k, w):
                        low = w & (-w)
                        e = lax.bitcast_convert_type(
                            low.astype(jnp.float32), jnp.int32)
                        r = g * 32 + (((e >> 23) & 255) - 127)
                        for j in range(D // 16):
                            sl = pl.ds(16 * j, 16)
                            acc_v[r, sl] = jnp.maximum(acc_v[r, sl], row[j])
                        return w & (w - 1)
                return 0

            lax.fori_loop(0, C, _point, 0)
        pltpu.sync_copy(acc_v, out_hbm.at[wid])

    return sc_seg_max


_sc_seg_max = _make_sc_seg_max()


def kernel(feature, feature_geo, xyz, center, plane_centers, plane_normals,
           plane_xyz_min, plane_xyz_max, W1, b1, W2, b2, W3, b3):
    f32 = jnp.float32
    clouds = xyz + center
    cl8 = jnp.zeros((N, 8), f32).at[:, :3].set(clouds)
    prm = jnp.zeros((16, P), f32)
    prm = prm.at[0, :].set(plane_xyz_min[:, 0])
    prm = prm.at[1, :].set(plane_xyz_max[:, 0])
    prm = prm.at[2, :].set(plane_xyz_min[:, 1])
    prm = prm.at[3, :].set(plane_xyz_max[:, 1])
    prm = prm.at[4:7, :].set(plane_centers.T)
    prm = prm.at[7:10, :].set(plane_normals.T)

    lanes = jnp.arange(NSEG)
    pw = jnp.zeros((NSEG, 16), f32)
    pw = pw.at[lanes, lanes // 16].set(2.0 ** (lanes % 16))
    pw = pw.at[lanes, 8 + lanes // 32].set(1.0)

    full = lambda a: pl.BlockSpec(a.shape, lambda i: (0,) * a.ndim)

    b1r = b1.reshape(1, D)
    b2r = b2.reshape(1, D)
    b3r = b3.reshape(1, 1)

    def run_half(h):
        off = h * GRIDH
        blk = lambda w: pl.BlockSpec((BN, w), lambda i: (i + off, 0))
        return pl.pallas_call(
            _tc_kernel,
            grid=(GRIDH,),
            in_specs=[
                blk(D),
                blk(D),
                blk(8),
                full(prm),
                full(W1), full(W2), full(W3),
                full(b1r), full(b2r), full(b3r), full(pw),
            ],
            out_specs=[
                pl.BlockSpec((BN, P), lambda i: (i, 0)),
                pl.BlockSpec((BN, D), lambda i: (i, 0)),
                pl.BlockSpec((BN, 16), lambda i: (i, 0)),
            ],
            out_shape=[
                jax.ShapeDtypeStruct((HALF, P), f32),
                jax.ShapeDtypeStruct((NPAD, D), f32),
                jax.ShapeDtypeStruct((NPAD, 16), jnp.int32),
            ],
            compiler_params=pltpu.CompilerParams(
                dimension_semantics=("arbitrary",),
            ),
        )(feature, feature_geo, cl8, prm, W1, W2, W3, b1r, b2r, b3r, pw)

    pl_a, h2_a, hw_a = run_half(0)
    acc_a = _sc_seg_max(h2_a, hw_a)
    pl_b, h2_b, hw_b = run_half(1)
    acc_b = _sc_seg_max(h2_b, hw_b)

    on_f, off_f = pl.pallas_call(
        _merge_kernel,
        in_specs=[pl.BlockSpec((NW, NSEG, D), lambda: (0, 0, 0)),
                  pl.BlockSpec((NW, NSEG, D), lambda: (0, 0, 0))],
        out_specs=[pl.BlockSpec((P, D), lambda: (0, 0)),
                   pl.BlockSpec((P, D), lambda: (0, 0))],
        out_shape=[jax.ShapeDtypeStruct((P, D), f32),
                   jax.ShapeDtypeStruct((P, D), f32)],
    )(acc_a, acc_b)

    pl_nt = jnp.concatenate([pl_a, pl_b], axis=0)
    return jnp.concatenate(
        [pl_nt.T.reshape(-1), on_f.reshape(-1), off_f.reshape(-1)])


# speedup vs baseline: 1.1818x; 1.1818x over previous
import functools

import jax
import jax.numpy as jnp
from jax import lax
from jax.experimental import pallas as pl
from jax.experimental.pallas import tpu as pltpu
from jax.experimental.pallas import tpu_sc as plsc

N = 50000
D = 128
P = 64
BN = 2000
NC = 2
NS = 16
NW = NC * NS
C = 192
CAP = 1728
NCH = CAP // C
NPAD = 56000
NSEG = 2 * P


def _tc_kernel(feat_ref, geo_ref, cl8_ref, prm_ref,
               w1_ref, w2_ref, w3_ref, b1_ref, b2_ref, b3_ref, pw_ref,
               pl_ref, h2_ref, hw_ref):
    c8 = cl8_ref[...]
    prm = prm_ref[...]
    x = c8[:, 0:1]
    y = c8[:, 1:2]
    z = c8[:, 2:3]
    rmask = ((x >= prm[0:1, :]) & (x < prm[1:2, :]) &
             (y >= prm[2:3, :]) & (y < prm[3:4, :]))
    bf = lambda a: a.astype(jnp.bfloat16).astype(jnp.float32)
    dx = bf(x - prm[4:5, :])
    dy = bf(y - prm[5:6, :])
    dz = bf(z - prm[6:7, :])
    dist = jnp.abs(dx * bf(prm[7:8, :]) + dy * bf(prm[8:9, :])
                   + dz * bf(prm[9:10, :]))
    mask = rmask & (dist < 0.1)

    fcat = jnp.concatenate([feat_ref[...], geo_ref[...]], axis=1)
    h1 = jnp.maximum(
        jnp.dot(fcat, w1_ref[...], preferred_element_type=jnp.float32)
        + b1_ref[...], 0.0)
    h2 = jnp.maximum(
        jnp.dot(h1, w2_ref[...], preferred_element_type=jnp.float32)
        + b2_ref[...], 0.0)
    logit = (jnp.dot(h2, w3_ref[...], preferred_element_type=jnp.float32)
             + b3_ref[...])

    pl_ref[...] = jnp.where(mask, logit, 0.0)
    h2_ref[...] = h2

    pos = logit > 0.0
    mo = jnp.concatenate([(mask & pos).astype(jnp.float32),
                          (mask & (~pos)).astype(jnp.float32)], axis=1)
    hw_f = jnp.dot(mo, pw_ref[...], preferred_element_type=jnp.float32)
    hw_ref[...] = hw_f.astype(jnp.int32)


def _merge_kernel(acc_ref, on_ref, off_ref):
    m = acc_ref[0]
    for t in range(1, NW):
        m = jnp.maximum(m, acc_ref[t])
    on_ref[...] = m[:P]
    off_ref[...] = m[P:]


def _make_sc_seg_max():
    mesh = plsc.VectorSubcoreMesh(core_axis_name="c", subcore_axis_name="s")

    @functools.partial(
        pl.kernel, mesh=mesh,
        out_type=jax.ShapeDtypeStruct((NW, NSEG, D), jnp.float32),
        scratch_types=[
            pltpu.VMEM((2, C, D), jnp.float32),
            pltpu.VMEM((2, C, 16), jnp.int32),
            pltpu.VMEM((NSEG, D), jnp.float32),
            pltpu.SemaphoreType.DMA((4,)),
        ],
    )
    def sc_seg_max(h2_hbm, hw_hbm, out_hbm, h2_v, hw_v, acc_v, sems):
        wid = lax.axis_index("s") * NC + lax.axis_index("c")
        base = wid * CAP

        def _zero_row(r, _):
            for j in range(D // 16):
                acc_v[r, pl.ds(16 * j, 16)] = jnp.zeros((16,), jnp.float32)
            return 0

        lax.fori_loop(0, NSEG, _zero_row, 0)

        def _issue(c, b):
            start = base + c * C
            return (pltpu.async_copy(h2_hbm.at[pl.ds(start, C)],
                                     h2_v.at[b], sems.at[2 * b]),
                    pltpu.async_copy(hw_hbm.at[pl.ds(start, C)],
                                     hw_v.at[b], sems.at[2 * b + 1]))

        pending = _issue(0, 0)
        for c in range(NCH):
            b = c % 2
            for h in pending:
                h.wait()
            if c + 1 < NCH:
                pending = _issue(c + 1, 1 - b)
            start = base + c * C

            def _point(ci, _):
                valid = (start + ci) < N
                row = [h2_v[b, ci, pl.ds(16 * j, 16)] for j in range(D // 16)]
                hwrow = hw_v[b, ci, pl.ds(0, 16)]

                for g in range(4):
                    w32 = hwrow[2 * g] | (hwrow[2 * g + 1] << 16)
                    w0 = jnp.where(valid, w32, 0)
                    cnt = jnp.where(valid, hwrow[8 + g], 0)

                    @pl.loop(0, cnt, init_carry=w0, unroll=False)
                    def _walk(k, w):
                        low = w & (-w)
                        e = lax.bitcast_convert_type(
                            low.astype(jnp.float32), jnp.int32)
                        r = g * 32 + (((e >> 23) & 255) - 127)
                        for j in range(D // 16):
                            sl = pl.ds(16 * j, 16)
                            acc_v[r, sl] = jnp.maximum(acc_v[r, sl], row[j])
                        return w & (w - 1)
                return 0

            lax.fori_loop(0, C, _point, 0)
        pltpu.sync_copy(acc_v, out_hbm.at[wid])

    return sc_seg_max


_sc_seg_max = _make_sc_seg_max()


def kernel(feature, feature_geo, xyz, center, plane_centers, plane_normals,
           plane_xyz_min, plane_xyz_max, W1, b1, W2, b2, W3, b3):
    f32 = jnp.float32
    clouds = xyz + center
    cl8 = jnp.zeros((N, 8), f32).at[:, :3].set(clouds)
    prm = jnp.zeros((16, P), f32)
    prm = prm.at[0, :].set(plane_xyz_min[:, 0])
    prm = prm.at[1, :].set(plane_xyz_max[:, 0])
    prm = prm.at[2, :].set(plane_xyz_min[:, 1])
    prm = prm.at[3, :].set(plane_xyz_max[:, 1])
    prm = prm.at[4:7, :].set(plane_centers.T)
    prm = prm.at[7:10, :].set(plane_normals.T)

    lanes = jnp.arange(NSEG)
    pw = jnp.zeros((NSEG, 16), f32)
    pw = pw.at[lanes, lanes // 16].set(2.0 ** (lanes % 16))
    pw = pw.at[lanes, 8 + lanes // 32].set(1.0)

    grid = (N // BN,)
    full = lambda a: pl.BlockSpec(a.shape, lambda i: (0,) * a.ndim)

    b1r = b1.reshape(1, D)
    b2r = b2.reshape(1, D)
    b3r = b3.reshape(1, 1)

    pl_nt, h2_arr, hw_arr = pl.pallas_call(
        _tc_kernel,
        grid=grid,
        in_specs=[
            pl.BlockSpec((BN, D), lambda i: (i, 0)),
            pl.BlockSpec((BN, D), lambda i: (i, 0)),
            pl.BlockSpec((BN, 8), lambda i: (i, 0)),
            full(prm),
            full(W1), full(W2), full(W3),
            full(b1r), full(b2r), full(b3r), full(pw),
        ],
        out_specs=[
            pl.BlockSpec((BN, P), lambda i: (i, 0)),
            pl.BlockSpec((BN, D), lambda i: (i, 0)),
            pl.BlockSpec((BN, 16), lambda i: (i, 0)),
        ],
        out_shape=[
            jax.ShapeDtypeStruct((N, P), f32),
            jax.ShapeDtypeStruct((NPAD, D), f32),
            jax.ShapeDtypeStruct((NPAD, 16), jnp.int32),
        ],
        compiler_params=pltpu.CompilerParams(
            dimension_semantics=("arbitrary",),
        ),
    )(feature, feature_geo, cl8, prm, W1, W2, W3, b1r, b2r, b3r, pw)

    accs = _sc_seg_max(h2_arr, hw_arr)

    on_f, off_f = pl.pallas_call(
        _merge_kernel,
        in_specs=[pl.BlockSpec((NW, NSEG, D), lambda: (0, 0, 0))],
        out_specs=[pl.BlockSpec((P, D), lambda: (0, 0)),
                   pl.BlockSpec((P, D), lambda: (0, 0))],
        out_shape=[jax.ShapeDtypeStruct((P, D), f32),
                   jax.ShapeDtypeStruct((P, D), f32)],
    )(accs)

    return jnp.concatenate(
        [pl_nt.T.reshape(-1), on_f.reshape(-1), off_f.reshape(-1)])


# speedup vs baseline: 1.1939x; 1.0102x over previous
import functools

import jax
import jax.numpy as jnp
from jax import lax
from jax.experimental import pallas as pl
from jax.experimental.pallas import tpu as pltpu
from jax.experimental.pallas import tpu_sc as plsc

N = 50000
D = 128
P = 64
BN = 2000
NC = 2
NS = 16
NW = NC * NS
C = 128
CAP = 1664
NCH = CAP // C
NPAD = 54000
NSEG = 2 * P


def _tc_kernel(feat_ref, geo_ref, cl8_ref, prm_ref,
               w1_ref, w2_ref, w3_ref, b1_ref, b2_ref, b3_ref, pw_ref,
               pl_ref, h2_ref, hw_ref):
    c8 = cl8_ref[...]
    prm = prm_ref[...]
    x = c8[:, 0:1]
    y = c8[:, 1:2]
    z = c8[:, 2:3]
    rmask = ((x >= prm[0:1, :]) & (x < prm[1:2, :]) &
             (y >= prm[2:3, :]) & (y < prm[3:4, :]))
    bf = lambda a: a.astype(jnp.bfloat16).astype(jnp.float32)
    dx = bf(x - prm[4:5, :])
    dy = bf(y - prm[5:6, :])
    dz = bf(z - prm[6:7, :])
    dist = jnp.abs(dx * bf(prm[7:8, :]) + dy * bf(prm[8:9, :])
                   + dz * bf(prm[9:10, :]))
    mask = rmask & (dist < 0.1)

    fcat = jnp.concatenate([feat_ref[...], geo_ref[...]], axis=1)
    h1 = jnp.maximum(
        jnp.dot(fcat, w1_ref[...], preferred_element_type=jnp.float32)
        + b1_ref[...], 0.0)
    h2 = jnp.maximum(
        jnp.dot(h1, w2_ref[...], preferred_element_type=jnp.float32)
        + b2_ref[...], 0.0)
    logit = (jnp.dot(h2, w3_ref[...], preferred_element_type=jnp.float32)
             + b3_ref[...])

    pl_ref[...] = jnp.where(mask, logit, 0.0)
    h2_ref[...] = h2

    pos = logit > 0.0
    mo = jnp.concatenate([(mask & pos).astype(jnp.float32),
                          (mask & (~pos)).astype(jnp.float32)], axis=1)
    hw_f = jnp.dot(mo, pw_ref[...], preferred_element_type=jnp.float32)
    hw_ref[...] = hw_f.astype(jnp.int32)


def _merge_kernel(acc_ref, on_ref, off_ref):
    m = acc_ref[0]
    for t in range(1, NW):
        m = jnp.maximum(m, acc_ref[t])
    on_ref[...] = m[:P]
    off_ref[...] = m[P:]


def _make_sc_seg_max():
    mesh = plsc.VectorSubcoreMesh(core_axis_name="c", subcore_axis_name="s")

    @functools.partial(
        pl.kernel, mesh=mesh,
        out_type=jax.ShapeDtypeStruct((NW, NSEG, D), jnp.float32),
        scratch_types=[
            pltpu.VMEM((2, C, D), jnp.float32),
            pltpu.VMEM((2, C, 16), jnp.int32),
            pltpu.VMEM((NSEG, D), jnp.float32),
            pltpu.SemaphoreType.DMA((4,)),
        ],
    )
    def sc_seg_max(h2_hbm, hw_hbm, out_hbm, h2_v, hw_v, acc_v, sems):
        wid = lax.axis_index("s") * NC + lax.axis_index("c")
        base = wid * CAP

        def _zero_row(r, _):
            for j in range(D // 16):
                acc_v[r, pl.ds(16 * j, 16)] = jnp.zeros((16,), jnp.float32)
            return 0

        lax.fori_loop(0, NSEG, _zero_row, 0)

        def _issue(c, b):
            start = base + c * C
            return (pltpu.async_copy(h2_hbm.at[pl.ds(start, C)],
                                     h2_v.at[b], sems.at[2 * b]),
                    pltpu.async_copy(hw_hbm.at[pl.ds(start, C)],
                                     hw_v.at[b], sems.at[2 * b + 1]))

        pending = _issue(0, 0)
        for c in range(NCH):
            b = c % 2
            for h in pending:
                h.wait()
            if c + 1 < NCH:
                pending = _issue(c + 1, 1 - b)
            start = base + c * C

            def _point(ci, _):
                valid = (start + ci) < N
                row = [h2_v[b, ci, pl.ds(16 * j, 16)] for j in range(D // 16)]
                hwrow = hw_v[b, ci, pl.ds(0, 16)]

                for g in range(4):
                    w32 = hwrow[2 * g] | (hwrow[2 * g + 1] << 16)
                    w0 = jnp.where(valid, w32, 0)
                    cnt = jnp.where(valid, hwrow[8 + g], 0)

                    @pl.loop(0, cnt, init_carry=w0, unroll=False)
                    def _walk(k, w):
                        low = w & (-w)
                        e = lax.bitcast_convert_type(
                            low.astype(jnp.float32), jnp.int32)
                        r = g * 32 + (((e >> 23) & 255) - 127)
                        for j in range(D // 16):
                            sl = pl.ds(16 * j, 16)
                            acc_v[r, sl] = jnp.maximum(acc_v[r, sl], row[j])
                        return w & (w - 1)
                return 0

            lax.fori_loop(0, C, _point, 0)
        pltpu.sync_copy(acc_v, out_hbm.at[wid])

    return sc_seg_max


_sc_seg_max = _make_sc_seg_max()


def kernel(feature, feature_geo, xyz, center, plane_centers, plane_normals,
           plane_xyz_min, plane_xyz_max, W1, b1, W2, b2, W3, b3):
    f32 = jnp.float32
    clouds = xyz + center
    cl8 = jnp.zeros((N, 8), f32).at[:, :3].set(clouds)
    prm = jnp.zeros((16, P), f32)
    prm = prm.at[0, :].set(plane_xyz_min[:, 0])
    prm = prm.at[1, :].set(plane_xyz_max[:, 0])
    prm = prm.at[2, :].set(plane_xyz_min[:, 1])
    prm = prm.at[3, :].set(plane_xyz_max[:, 1])
    prm = prm.at[4:7, :].set(plane_centers.T)
    prm = prm.at[7:10, :].set(plane_normals.T)

    lanes = jnp.arange(NSEG)
    pw = jnp.zeros((NSEG, 16), f32)
    pw = pw.at[lanes, lanes // 16].set(2.0 ** (lanes % 16))
    pw = pw.at[lanes, 8 + lanes // 32].set(1.0)

    grid = (N // BN,)
    full = lambda a: pl.BlockSpec(a.shape, lambda i: (0,) * a.ndim)

    b1r = b1.reshape(1, D)
    b2r = b2.reshape(1, D)
    b3r = b3.reshape(1, 1)

    pl_nt, h2_arr, hw_arr = pl.pallas_call(
        _tc_kernel,
        grid=grid,
        in_specs=[
            pl.BlockSpec((BN, D), lambda i: (i, 0)),
            pl.BlockSpec((BN, D), lambda i: (i, 0)),
            pl.BlockSpec((BN, 8), lambda i: (i, 0)),
            full(prm),
            full(W1), full(W2), full(W3),
            full(b1r), full(b2r), full(b3r), full(pw),
        ],
        out_specs=[
            pl.BlockSpec((BN, P), lambda i: (i, 0)),
            pl.BlockSpec((BN, D), lambda i: (i, 0)),
            pl.BlockSpec((BN, 16), lambda i: (i, 0)),
        ],
        out_shape=[
            jax.ShapeDtypeStruct((N, P), f32),
            jax.ShapeDtypeStruct((NPAD, D), f32),
            jax.ShapeDtypeStruct((NPAD, 16), jnp.int32),
        ],
        compiler_params=pltpu.CompilerParams(
            dimension_semantics=("arbitrary",),
        ),
    )(feature, feature_geo, cl8, prm, W1, W2, W3, b1r, b2r, b3r, pw)

    accs = _sc_seg_max(h2_arr, hw_arr)

    on_f, off_f = pl.pallas_call(
        _merge_kernel,
        in_specs=[pl.BlockSpec((NW, NSEG, D), lambda: (0, 0, 0))],
        out_specs=[pl.BlockSpec((P, D), lambda: (0, 0)),
                   pl.BlockSpec((P, D), lambda: (0, 0))],
        out_shape=[jax.ShapeDtypeStruct((P, D), f32),
                   jax.ShapeDtypeStruct((P, D), f32)],
    )(accs)

    return jnp.concatenate(
        [pl_nt.T.reshape(-1), on_f.reshape(-1), off_f.reshape(-1)])
